# baseline (speedup 1.0000x reference)
import jax
import jax.numpy as jnp
from jax.experimental import pallas as pl
from jax.experimental.pallas import tpu as pltpu

_B = 16
_C = 64
_HW = 1024
_K = 1024
_NUMEL = _B * _C * _HW
_NTOK = _B * _HW
_PB = 4


def _one_slab(X, W2_16, wn, iota_kf):
    M2 = jnp.dot(W2_16, X.astype(jnp.bfloat16),
                 preferred_element_type=jnp.float32)
    xn = jnp.sum(X * X, axis=0)
    T = (xn[None, :] + wn) - M2
    minv = jnp.min(T, axis=0)
    idxf = jnp.min(jnp.where(T == minv[None, :], iota_kf, float(_K)), axis=0)
    idx = idxf.astype(jnp.int32)
    E16 = jnp.where(iota_kf == idxf[None, :], 0.5, 0.0).astype(jnp.bfloat16)
    Q = jax.lax.dot_general(W2_16, E16, (((0,), (0,)), ((), ())),
                            preferred_element_type=jnp.float32)
    return idx, Q, E16, jnp.sum(minv)


def _vq_body(x_ref, w_ref, idx_ref, q_ref, loss_ref, perp_ref, esum_ref,
             acc_ref, iota_ref, w16_ref, wn_ref):
    b = pl.program_id(0)

    @pl.when(b == 0)
    def _init():
        W = w_ref[...]
        w16_ref[...] = W.astype(jnp.bfloat16) * jnp.bfloat16(2.0)
        wn_ref[...] = jnp.sum(W * W, axis=1, keepdims=True)
        iota_ref[...] = jax.lax.broadcasted_iota(
            jnp.int32, (_K, _HW), 0).astype(jnp.float32)
        esum_ref[...] = jnp.zeros_like(esum_ref)
        acc_ref[0] = 0.0

    W2_16 = w16_ref[...]
    wn = wn_ref[...]
    iota_kf = iota_ref[...]
    acc = acc_ref[0]
    for s in range(_PB):
        idx, Q, E16, msum = _one_slab(x_ref[s], W2_16, wn, iota_kf)
        idx_ref[s, 0, :] = idx
        q_ref[s] = Q
        esum_ref[...] += E16
        acc = acc + msum
    acc_ref[0] = acc

    @pl.when(b == _B // _PB - 1)
    def _finalize():
        loss_ref[...] = jnp.full((1, 1), acc_ref[0] * (1.25 / _NUMEL),
                                 jnp.float32)
        counts = jnp.sum(esum_ref[...].astype(jnp.float32), axis=1)
        p = counts * (2.0 / _NTOK)
        perp_ref[...] = jnp.full((1, 1),
                                 jnp.exp(-jnp.sum(p * jnp.log(p + 1e-10))),
                                 jnp.float32)


def kernel(inputs, W):
    x3 = inputs.reshape(_B, _C, _HW)
    idx3, q3, loss11, perp11 = pl.pallas_call(
        _vq_body,
        grid=(_B // _PB,),
        in_specs=[pl.BlockSpec((_PB, _C, _HW), lambda b: (b, 0, 0)),
                  pl.BlockSpec((_K, _C), lambda b: (0, 0))],
        out_specs=[pl.BlockSpec((_PB, 1, _HW), lambda b: (b, 0, 0)),
                   pl.BlockSpec((_PB, _C, _HW), lambda b: (b, 0, 0)),
                   pl.BlockSpec((1, 1), lambda b: (0, 0)),
                   pl.BlockSpec((1, 1), lambda b: (0, 0))],
        out_shape=[jax.ShapeDtypeStruct((_B, 1, _HW), jnp.int32),
                   jax.ShapeDtypeStruct((_B, _C, _HW), jnp.float32),
                   jax.ShapeDtypeStruct((1, 1), jnp.float32),
                   jax.ShapeDtypeStruct((1, 1), jnp.float32)],
        scratch_shapes=[pltpu.VMEM((_K, _HW), jnp.bfloat16),
                        pltpu.SMEM((1,), jnp.float32),
                        pltpu.VMEM((_K, _HW), jnp.float32),
                        pltpu.VMEM((_K, _C), jnp.bfloat16),
                        pltpu.VMEM((_K, 1), jnp.float32)],
    )(x3, W)
    loss = loss11[0, 0]
    perplexity = perp11[0, 0]
    quantized_out = q3.reshape(inputs.shape)
    codebook_indices = idx3.reshape(-1)
    return (loss, quantized_out, perplexity, codebook_indices)
